# baseline (speedup 1.0000x reference)
import jax
import jax.numpy as jnp
from jax import lax
from jax.experimental import pallas as pl
from jax.experimental.pallas import tpu as pltpu
from jax.experimental.pallas import tpu_sc as plsc

_BATCH = 16384
_N_FIELDS = 26
_VOCAB = 100000
_EMB = 16

_NC = 2
_NS = 16
_NW = _NC * _NS

_ROWS_PER_W = _BATCH // _NW
_IDX_PER_DMA = 128
_DMAS_PER_F = _ROWS_PER_W // _IDX_PER_DMA
_NBUF = 4


def _body(tab_hbm, xt_hbm, out_hbm, idx_v, b0, b1, b2, b3,
          g0, g1, g2, g3, o0, o1, o2, o3):
    bufs = (b0, b1, b2, b3)
    gsems = (g0, g1, g2, g3)
    osems = (o0, o1, o2, o3)

    wid = lax.axis_index("s") * _NC + lax.axis_index("c")
    base = wid * _ROWS_PER_W

    pltpu.sync_copy(xt_hbm.at[:, pl.ds(base, _ROWS_PER_W)], idx_v)

    gds = [None] * _N_FIELDS
    ods = [None] * _N_FIELDS

    def fire_gathers(f):
        b = f % _NBUF
        ds = []
        for j in range(_DMAS_PER_F):
            isl = idx_v.at[f, pl.ds(j * _IDX_PER_DMA, _IDX_PER_DMA)]
            dst = bufs[b].at[pl.ds(j * _IDX_PER_DMA, _IDX_PER_DMA), :]
            ds.append(pltpu.async_copy(tab_hbm.at[f].at[isl], dst, gsems[b]))
        gds[f] = ds

    def drain_and_out(f):
        b = f % _NBUF
        for d in gds[f]:
            d.wait()
        dst = out_hbm.at[pl.ds(base, _ROWS_PER_W), pl.ds(f * _EMB, _EMB)]
        ods[f] = pltpu.async_copy(bufs[b], dst, osems[b])

    fire_gathers(0)
    for f in range(1, _N_FIELDS + 1):
        if f < _N_FIELDS:
            if f >= _NBUF:
                ods[f - _NBUF].wait()
            fire_gathers(f)
        drain_and_out(f - 1)
    for f in range(_N_FIELDS - _NBUF, _N_FIELDS):
        ods[f].wait()


_gather_call = pl.kernel(
    _body,
    out_type=jax.ShapeDtypeStruct((_BATCH, _N_FIELDS * _EMB), jnp.float32),
    mesh=plsc.VectorSubcoreMesh(core_axis_name="c", subcore_axis_name="s",
                                num_cores=_NC, num_subcores=_NS),
    scratch_types=(
        [pltpu.VMEM((_N_FIELDS, _ROWS_PER_W), jnp.int32)]
        + [pltpu.VMEM((_ROWS_PER_W, _EMB), jnp.float32) for _ in range(_NBUF)]
        + [pltpu.SemaphoreType.DMA for _ in range(2 * _NBUF)]
    ),
    compiler_params=pltpu.CompilerParams(use_tc_tiling_on_sc=False),
)


def kernel(x, tables):
    xt = x.astype(jnp.int32).T
    return _gather_call(tables, xt)
